# baseline (speedup 1.0000x reference)
import jax
import jax.numpy as jnp
from jax.experimental import pallas as pl
from jax.experimental.pallas import tpu as pltpu

_BM = 512
_NCHUNK = 8


def _mm_kernel(x_ref, w_hbm, b_ref, o_ref, w_vmem, sems):
    i = pl.program_id(0)
    K = w_vmem.shape[0]
    ck = K // _NCHUNK

    @pl.when(i == 0)
    def _():
        for c in range(_NCHUNK):
            pltpu.make_async_copy(
                w_hbm.at[pl.ds(c * ck, ck), :],
                w_vmem.at[pl.ds(c * ck, ck), :],
                sems.at[c],
            ).start()
        acc = b_ref[...].astype(jnp.float32)
        for c in range(_NCHUNK):
            pltpu.make_async_copy(
                w_hbm.at[pl.ds(c * ck, ck), :],
                w_vmem.at[pl.ds(c * ck, ck), :],
                sems.at[c],
            ).wait()
            acc = acc + jnp.dot(
                x_ref[:, c * ck : (c + 1) * ck],
                w_vmem[pl.ds(c * ck, ck), :],
                preferred_element_type=jnp.float32,
            )
        o_ref[...] = acc

    @pl.when(i != 0)
    def _():
        acc = jnp.dot(x_ref[...], w_vmem[...], preferred_element_type=jnp.float32)
        o_ref[...] = acc + b_ref[...]


def kernel(input, weight, bias):
    M, K = input.shape
    _, N = weight.shape
    bias2d = bias.reshape(1, N)
    return pl.pallas_call(
        _mm_kernel,
        grid=(M // _BM,),
        in_specs=[
            pl.BlockSpec((_BM, K), lambda i: (i, 0)),
            pl.BlockSpec(memory_space=pltpu.MemorySpace.HBM),
            pl.BlockSpec((1, N), lambda i: (0, 0)),
        ],
        out_specs=pl.BlockSpec((_BM, N), lambda i: (i, 0)),
        out_shape=jax.ShapeDtypeStruct((M, N), jnp.float32),
        scratch_shapes=[
            pltpu.VMEM((K, N), jnp.float32),
            pltpu.SemaphoreType.DMA((_NCHUNK,)),
        ],
        compiler_params=pltpu.CompilerParams(vmem_limit_bytes=100 * 1024 * 1024),
    )(input, weight, bias2d)
